# baseline (device time: 29969 ns/iter reference)
import jax
import jax.numpy as jnp
from jax import lax
from jax.experimental import pallas as pl
from jax.experimental.pallas import tpu as pltpu

N_CHUNK = 2


def kernel(x, router, W1, W2):
    t_loc, d = x.shape
    e_loc, _, f = W1.shape
    t_chk = t_loc // N_CHUNK

    def body(x_ref, r_ref, w1_ref, w2_ref, out_ref,
             xfull_ref, rpeer_ref, wsend_ref, wrecv_ref,
             csend_ref, b1recv_ref, dsend_ref, b2recv_ref,
             send_sems, recv_sems):
        my_x = lax.axis_index("x")
        my_y = lax.axis_index("y")
        xpeer = (1 - my_x, my_y)
        ypeer = (my_x, 1 - my_y)

        def exchange(src, dst, sem, peer):
            return pltpu.make_async_remote_copy(
                src_ref=src, dst_ref=dst,
                send_sem=send_sems.at[sem], recv_sem=recv_sems.at[sem],
                device_id=peer, device_id_type=pl.DeviceIdType.MESH,
            )

        def pick(a2, idx):
            return jnp.where(idx == 0, a2[:, 0:1], a2[:, 1:2])

        barrier_sem = pltpu.get_barrier_semaphore()
        for nbr in (xpeer, ypeer):
            pl.semaphore_signal(barrier_sem, inc=1, device_id=nbr,
                                device_id_type=pl.DeviceIdType.MESH)
        pl.semaphore_wait(barrier_sem, 2)

        rdma_r = exchange(r_ref, rpeer_ref, 0, xpeer)
        rdma_r.start()

        xmb = x_ref[...].astype(jnp.bfloat16)
        xfull_ref[pl.ds(0, t_loc), :] = xmb

        rdma_r.wait()

        rfull = jnp.concatenate([r_ref[...], rpeer_ref[...]], axis=-1)
        g = jnp.dot(x_ref[...], rfull,
                    precision=lax.Precision.HIGHEST,
                    preferred_element_type=jnp.float32)
        m1 = jnp.max(g, axis=-1, keepdims=True)
        m2 = jnp.max(jnp.where(g == m1, -1e30, g), axis=-1, keepdims=True)
        sel = g >= m2
        ex = jnp.where(sel, jnp.exp(g - m1), 0.0)
        w_mine = ex / jnp.sum(ex, axis=-1, keepdims=True)

        wsend_ref[...] = w_mine[:, e_loc:]
        rdma_x = exchange(xfull_ref.at[pl.ds(0, t_loc), :],
                          xfull_ref.at[pl.ds(t_loc, t_loc), :], 1, xpeer)
        rdma_x.start()
        rdma_w = exchange(wsend_ref, wrecv_ref, 2, xpeer)
        rdma_w.start()

        w1b = jnp.where(my_y == 0, w1_ref[0], w1_ref[1]).astype(jnp.bfloat16)
        w2b = jnp.where(my_y == 0, w2_ref[0], w2_ref[1]).astype(jnp.bfloat16)

        def expert(xb, wcol):
            h = jnp.maximum(
                jnp.dot(xb, w1b, preferred_element_type=jnp.float32), 0.0)
            p = jnp.dot(h.astype(jnp.bfloat16), w2b,
                        preferred_element_type=jnp.float32)
            return wcol * p

        c_mine = expert(xmb, pick(w_mine[:, :e_loc], my_y))

        rdma_x.wait()
        rdma_w.wait()

        wp_col = pick(wrecv_ref[...], my_y)
        rdma_b1 = []
        for c in range(N_CHUNK):
            lo = c * t_chk
            xpc = xfull_ref[pl.ds(t_loc + lo, t_chk), :]
            cc = expert(xpc, wp_col[lo:lo + t_chk, :])
            csend_ref[c, :, :] = cc.astype(jnp.bfloat16)
            rc = exchange(csend_ref.at[c], b1recv_ref.at[c], 3 + c, xpeer)
            rc.start()
            rdma_b1.append(rc)

        rdma_b2 = []
        dparts = []
        for c in range(N_CHUNK):
            lo = c * t_chk
            rdma_b1[c].wait()
            dc = c_mine[lo:lo + t_chk, :] + \
                b1recv_ref[c, :, :].astype(jnp.float32)
            dparts.append(dc)
            dsend_ref[c, :, :] = dc.astype(jnp.bfloat16)
            rc = exchange(dsend_ref.at[c], b2recv_ref.at[c],
                          3 + N_CHUNK + c, ypeer)
            rc.start()
            rdma_b2.append(rc)

        for c in range(N_CHUNK):
            lo = c * t_chk
            rdma_b2[c].wait()
            out_ref[pl.ds(lo, t_chk), :] = \
                dparts[c] + b2recv_ref[c, :, :].astype(jnp.float32)

    n_sems = 3 + 2 * N_CHUNK
    return pl.pallas_call(
        body,
        out_shape=jax.ShapeDtypeStruct((t_loc, d), jnp.float32),
        in_specs=[pl.BlockSpec(memory_space=pltpu.VMEM)] * 4,
        out_specs=pl.BlockSpec(memory_space=pltpu.VMEM),
        scratch_shapes=[
            pltpu.VMEM((2 * t_loc, d), jnp.bfloat16),
            pltpu.VMEM((d, e_loc), jnp.float32),
            pltpu.VMEM((t_loc, e_loc), jnp.float32),
            pltpu.VMEM((t_loc, e_loc), jnp.float32),
            pltpu.VMEM((N_CHUNK, t_chk, d), jnp.bfloat16),
            pltpu.VMEM((N_CHUNK, t_chk, d), jnp.bfloat16),
            pltpu.VMEM((N_CHUNK, t_chk, d), jnp.bfloat16),
            pltpu.VMEM((N_CHUNK, t_chk, d), jnp.bfloat16),
            pltpu.SemaphoreType.DMA((n_sems,)),
            pltpu.SemaphoreType.DMA((n_sems,)),
        ],
        compiler_params=pltpu.CompilerParams(collective_id=0),
    )(x, router, W1, W2)


# device time: 27412 ns/iter; 1.0933x vs baseline; 1.0933x over previous
import jax
import jax.numpy as jnp
from jax import lax
from jax.experimental import pallas as pl
from jax.experimental.pallas import tpu as pltpu

N_CHUNK = 2


def kernel(x, router, W1, W2):
    t_loc, d = x.shape
    e_loc, _, f = W1.shape
    t_chk = t_loc // N_CHUNK

    def body(x_ref, r_ref, w1_hbm, w2_hbm, out_ref,
             w1v_ref, w2v_ref, xfull_ref, rpeer_ref, wsend_ref, wrecv_ref,
             csend_ref, b1recv_ref, send_sems, recv_sems, copy_sems):
        my_x = lax.axis_index("x")
        my_y = lax.axis_index("y")
        peer = (1 - my_x, my_y)

        def exchange(src, dst, sem):
            return pltpu.make_async_remote_copy(
                src_ref=src, dst_ref=dst,
                send_sem=send_sems.at[sem], recv_sem=recv_sems.at[sem],
                device_id=peer, device_id_type=pl.DeviceIdType.MESH,
            )

        cp1 = pltpu.make_async_copy(w1_hbm, w1v_ref, copy_sems.at[0])
        cp2 = pltpu.make_async_copy(w2_hbm, w2v_ref, copy_sems.at[1])
        cp1.start()
        cp2.start()

        barrier_sem = pltpu.get_barrier_semaphore()
        pl.semaphore_signal(barrier_sem, inc=1, device_id=peer,
                            device_id_type=pl.DeviceIdType.MESH)
        pl.semaphore_wait(barrier_sem, 1)

        xmb = x_ref[...].astype(jnp.bfloat16)
        xfull_ref[pl.ds(0, t_loc), :] = xmb
        rdma_x = exchange(xfull_ref.at[pl.ds(0, t_loc), :],
                          xfull_ref.at[pl.ds(t_loc, t_loc), :], 1)
        rdma_x.start()
        rdma_r = exchange(r_ref, rpeer_ref, 0)
        rdma_r.start()

        cp1.wait()
        cp2.wait()
        w1b = [w1v_ref[j].astype(jnp.bfloat16) for j in range(e_loc)]
        w2b = [w2v_ref[j].astype(jnp.bfloat16) for j in range(e_loc)]

        def expert(xb, j):
            h = jnp.maximum(
                jnp.dot(xb, w1b[j], preferred_element_type=jnp.float32), 0.0)
            return jnp.dot(h.astype(jnp.bfloat16), w2b[j],
                           preferred_element_type=jnp.float32)

        p_mine = [expert(xmb, j) for j in range(e_loc)]

        rdma_r.wait()
        rfull = jnp.concatenate([r_ref[...], rpeer_ref[...]], axis=-1)
        g = jnp.dot(x_ref[...], rfull,
                    precision=lax.Precision.HIGHEST,
                    preferred_element_type=jnp.float32)
        m1 = jnp.max(g, axis=-1, keepdims=True)
        m2 = jnp.max(jnp.where(g == m1, -1e30, g), axis=-1, keepdims=True)
        sel = g >= m2
        ex = jnp.where(sel, jnp.exp(g - m1), 0.0)
        w_mine = ex / jnp.sum(ex, axis=-1, keepdims=True)

        wsend_ref[...] = w_mine[:, e_loc:]
        rdma_w = exchange(wsend_ref, wrecv_ref, 2)
        rdma_w.start()

        c_mine = w_mine[:, 0:1] * p_mine[0] + w_mine[:, 1:2] * p_mine[1]

        rdma_x.wait()
        rdma_w.wait()

        wp = wrecv_ref[...]
        rdma_b1 = []
        for c in range(N_CHUNK):
            lo = c * t_chk
            xpc = xfull_ref[pl.ds(t_loc + lo, t_chk), :]
            cc = wp[lo:lo + t_chk, 0:1] * expert(xpc, 0) + \
                wp[lo:lo + t_chk, 1:2] * expert(xpc, 1)
            csend_ref[c, :, :] = cc.astype(jnp.bfloat16)
            rc = exchange(csend_ref.at[c], b1recv_ref.at[c], 3 + c)
            rc.start()
            rdma_b1.append(rc)

        for c in range(N_CHUNK):
            lo = c * t_chk
            rdma_b1[c].wait()
            out_ref[pl.ds(lo, t_chk), :] = \
                c_mine[lo:lo + t_chk, :] + \
                b1recv_ref[c, :, :].astype(jnp.float32)

    n_sems = 3 + N_CHUNK
    return pl.pallas_call(
        body,
        out_shape=jax.ShapeDtypeStruct((t_loc, d), jnp.float32),
        in_specs=[
            pl.BlockSpec(memory_space=pltpu.MemorySpace.VMEM),
            pl.BlockSpec(memory_space=pltpu.MemorySpace.VMEM),
            pl.BlockSpec(memory_space=pltpu.MemorySpace.HBM),
            pl.BlockSpec(memory_space=pltpu.MemorySpace.HBM),
        ],
        out_specs=pl.BlockSpec(memory_space=pltpu.MemorySpace.VMEM),
        scratch_shapes=[
            pltpu.VMEM((e_loc, d, f), jnp.float32),
            pltpu.VMEM((e_loc, f, d), jnp.float32),
            pltpu.VMEM((2 * t_loc, d), jnp.bfloat16),
            pltpu.VMEM((d, e_loc), jnp.float32),
            pltpu.VMEM((t_loc, e_loc), jnp.float32),
            pltpu.VMEM((t_loc, e_loc), jnp.float32),
            pltpu.VMEM((N_CHUNK, t_chk, d), jnp.bfloat16),
            pltpu.VMEM((N_CHUNK, t_chk, d), jnp.bfloat16),
            pltpu.SemaphoreType.DMA((n_sems,)),
            pltpu.SemaphoreType.DMA((n_sems,)),
            pltpu.SemaphoreType.DMA((2,)),
        ],
        compiler_params=pltpu.CompilerParams(collective_id=0),
    )(x, router, W1, W2)
